# Initial kernel scaffold; baseline (speedup 1.0000x reference)
import jax
import jax.numpy as jnp
from jax.experimental import pallas as pl


def kernel(x, embedding):
    raise NotImplementedError("write your pallas kernel here")


# speedup vs baseline: 1.6324x; 1.6324x over previous
import functools

import jax
import jax.numpy as jnp
from jax.experimental import pallas as pl
from jax.experimental.pallas import tpu as pltpu

PRECISION = 10


def _block_kernel(x_ref, emb_ref, o_ref, *, seq_block: int, precision: int):
    s0 = pl.program_id(1) * seq_block
    rows = jax.lax.broadcasted_iota(jnp.int32, (seq_block, precision), 0) + s0
    phases = jax.lax.broadcasted_iota(jnp.int32, (seq_block, precision), 1)
    one_hot = (rows % precision == phases).astype(jnp.float32)
    emb_block = jnp.dot(one_hot, emb_ref[...],
                        preferred_element_type=jnp.float32)
    o_ref[...] = x_ref[...] + emb_block[None, :, :]


def kernel(x, embedding):
    batch, seq_len, d_model = x.shape
    precision = embedding.shape[0]
    seq_block = 512
    grid = (batch, seq_len // seq_block)
    fn = pl.pallas_call(
        functools.partial(_block_kernel, seq_block=seq_block,
                          precision=precision),
        grid=grid,
        in_specs=[
            pl.BlockSpec((1, seq_block, d_model), lambda b, s: (b, s, 0)),
            pl.BlockSpec((precision, d_model), lambda b, s: (0, 0)),
        ],
        out_specs=pl.BlockSpec((1, seq_block, d_model), lambda b, s: (b, s, 0)),
        out_shape=jax.ShapeDtypeStruct(x.shape, x.dtype),
    )
    return fn(x, embedding.astype(jnp.float32))
